# baseline (speedup 1.0000x reference)
import functools

import jax
import jax.numpy as jnp
from jax import lax
from jax.experimental import pallas as pl
from jax.experimental.pallas import tpu as pltpu
from jax.experimental.pallas import tpu_sc as plsc

_CB = 8192
_D = 32
_MBLK = 1024
_KBLK = 1024
_NKB = _CB // _KBLK


def _dist_body(x_ref, et_ref, ind_ref, loss_ref):
    x = x_ref[...]
    x2 = jnp.sum(x * x, axis=1)

    xb = x.astype(jnp.bfloat16)

    def body(c, carry):
        run_max, run_idx = carry
        et_c = et_ref[:, pl.ds(c * _KBLK, _KBLK)]
        s = jnp.dot(xb, et_c.astype(jnp.bfloat16),
                    preferred_element_type=jnp.float32)
        e2 = jnp.sum(et_c * et_c, axis=0)
        score = -(x2[:, None] - 2.0 * s + e2[None, :])
        cmax = jnp.max(score, axis=1)
        cidx = jnp.argmax(score, axis=1).astype(jnp.int32)
        upd = cmax > run_max
        run_idx = jnp.where(upd, cidx + c * _KBLK, run_idx)
        run_max = jnp.where(upd, cmax, run_max)
        return run_max, run_idx

    def half_argmax(lo, hi):
        init = (jnp.full((_MBLK,), -jnp.inf, dtype=jnp.float32),
                jnp.zeros((_MBLK,), dtype=jnp.int32))
        return lax.fori_loop(lo, hi, body, init)

    m1, i1 = half_argmax(0, _NKB // 2)
    m2, i2 = half_argmax(_NKB // 2, _NKB)
    m1b = m1.astype(jnp.bfloat16).astype(jnp.float32)
    keep1 = m1b >= m2
    run_idx = jnp.where(keep1, i1, i2)
    run_max = jnp.where(keep1, m1, m2)
    ind_ref[...] = run_idx

    sqd = -run_max
    i = pl.program_id(0)
    nprog = pl.num_programs(0)
    prev = jnp.where(i == 0, 0.0, loss_ref[0, 0])
    tot = prev + jnp.sum(sqd)
    denom = jnp.float32(nprog * _MBLK * _D)
    loss_ref[0, 0] = jnp.where(i == nprog - 1, tot / denom, tot)


def _argmin_and_loss(flat, embed_t):
    m = flat.shape[0]
    return pl.pallas_call(
        _dist_body,
        grid=(m // _MBLK,),
        in_specs=[
            pl.BlockSpec((_MBLK, _D), lambda i: (i, 0)),
            pl.BlockSpec((_D, _CB), lambda i: (0, 0)),
        ],
        out_specs=[
            pl.BlockSpec((_MBLK,), lambda i: (i,)),
            pl.BlockSpec(memory_space=pltpu.SMEM),
        ],
        out_shape=[
            jax.ShapeDtypeStruct((m,), jnp.int32),
            jax.ShapeDtypeStruct((1, 1), jnp.float32),
        ],
    )(flat, embed_t)


_IBLK = 128


@functools.cache
def _sc_gather(b_total):
    info = plsc.get_sparse_core_info()
    nc, ns = info.num_cores, info.num_subcores
    nw = nc * ns
    b_per_w = b_total // nw
    nj = b_per_w // _IBLK
    mesh = plsc.VectorSubcoreMesh(core_axis_name="c", subcore_axis_name="s")

    @functools.partial(
        pl.kernel, mesh=mesh,
        compiler_params=pltpu.CompilerParams(use_tc_tiling_on_sc=False),
        out_type=jax.ShapeDtypeStruct((b_total, _D), jnp.float32),
        scratch_types=[
            pltpu.VMEM((nj, _IBLK), jnp.int32),
            pltpu.VMEM((b_per_w, _D), jnp.float32),
            pltpu.SemaphoreType.DMA,
        ],
    )
    def gather(table_hbm, idx_hbm, out_hbm, idx_v, rows_v, sem):
        wid = lax.axis_index("s") * nc + lax.axis_index("c")
        pltpu.sync_copy(idx_hbm.at[pl.ds(wid * nj, nj)], idx_v)
        copies = [
            pltpu.async_copy(table_hbm.at[idx_v.at[j]],
                             rows_v.at[pl.ds(j * _IBLK, _IBLK)], sem)
            for j in range(nj)
        ]
        for c in copies:
            c.wait()
        pltpu.sync_copy(rows_v, out_hbm.at[pl.ds(wid * b_per_w, b_per_w)])

    return gather


def kernel(x, embed):
    b, n, d = x.shape
    flat = x.reshape(-1, d)
    ind, loss = _argmin_and_loss(flat, embed.T)
    quant = _sc_gather(flat.shape[0])(embed, ind.reshape(-1, _IBLK))
    return quant.reshape(b, n, d), ind.reshape(b, n), loss[0, 0]


# speedup vs baseline: 1.2728x; 1.2728x over previous
import functools

import jax
import jax.numpy as jnp
from jax import lax
from jax.experimental import pallas as pl
from jax.experimental.pallas import tpu as pltpu
from jax.experimental.pallas import tpu_sc as plsc

_CB = 8192
_D = 32
_MBLK = 1024
_KBLK = 1024
_NKB = _CB // _KBLK


def _dist_body(x_ref, et_ref, ind_ref, loss_ref):
    x = x_ref[...]
    x2 = jnp.sum(x * x, axis=1)

    xb2 = (2.0 * x).astype(jnp.bfloat16)

    def chunk(c):
        et_c = et_ref[:, c * _KBLK:(c + 1) * _KBLK]
        s = jnp.dot(xb2, et_c.astype(jnp.bfloat16),
                    preferred_element_type=jnp.float32)
        e2_c = jnp.sum(et_c * et_c, axis=0)
        q = (x2[:, None] - s) + e2_c[None, :]
        cmin = jnp.min(q, axis=1)
        cidx = jnp.argmin(q, axis=1).astype(jnp.int32) + c * _KBLK
        return cmin, cidx

    def half_argmin(lo, hi):
        run_min, run_idx = chunk(lo)
        for c in range(lo + 1, hi):
            cmin, cidx = chunk(c)
            upd = cmin < run_min
            run_idx = jnp.where(upd, cidx, run_idx)
            run_min = jnp.where(upd, cmin, run_min)
        return run_min, run_idx

    m1, i1 = half_argmin(0, _NKB // 2)
    m2, i2 = half_argmin(_NKB // 2, _NKB)
    m1b = m1.astype(jnp.bfloat16).astype(jnp.float32)
    keep1 = m1b <= m2
    run_idx = jnp.where(keep1, i1, i2)
    run_min = jnp.where(keep1, m1, m2)
    ind_ref[...] = run_idx

    sqd = run_min
    i = pl.program_id(0)
    nprog = pl.num_programs(0)
    prev = jnp.where(i == 0, 0.0, loss_ref[0, 0])
    tot = prev + jnp.sum(sqd)
    denom = jnp.float32(nprog * _MBLK * _D)
    loss_ref[0, 0] = jnp.where(i == nprog - 1, tot / denom, tot)


def _argmin_and_loss(flat, embed_t):
    m = flat.shape[0]
    return pl.pallas_call(
        _dist_body,
        grid=(m // _MBLK,),
        in_specs=[
            pl.BlockSpec((_MBLK, _D), lambda i: (i, 0)),
            pl.BlockSpec((_D, _CB), lambda i: (0, 0)),
        ],
        out_specs=[
            pl.BlockSpec((_MBLK,), lambda i: (i,)),
            pl.BlockSpec(memory_space=pltpu.SMEM),
        ],
        out_shape=[
            jax.ShapeDtypeStruct((m,), jnp.int32),
            jax.ShapeDtypeStruct((1, 1), jnp.float32),
        ],
    )(flat, embed_t)


_IBLK = 128


@functools.cache
def _sc_gather(b_total):
    info = plsc.get_sparse_core_info()
    nc, ns = info.num_cores, info.num_subcores
    nw = nc * ns
    b_per_w = b_total // nw
    nj = b_per_w // _IBLK
    mesh = plsc.VectorSubcoreMesh(core_axis_name="c", subcore_axis_name="s")

    @functools.partial(
        pl.kernel, mesh=mesh,
        compiler_params=pltpu.CompilerParams(use_tc_tiling_on_sc=False),
        out_type=jax.ShapeDtypeStruct((b_total, _D), jnp.float32),
        scratch_types=[
            pltpu.VMEM((nj, _IBLK), jnp.int32),
            pltpu.VMEM((b_per_w, _D), jnp.float32),
            pltpu.SemaphoreType.DMA,
        ],
    )
    def gather(table_hbm, idx_hbm, out_hbm, idx_v, rows_v, sem):
        wid = lax.axis_index("s") * nc + lax.axis_index("c")
        pltpu.sync_copy(idx_hbm.at[pl.ds(wid * nj, nj)], idx_v)
        copies = [
            pltpu.async_copy(table_hbm.at[idx_v.at[j]],
                             rows_v.at[pl.ds(j * _IBLK, _IBLK)], sem)
            for j in range(nj)
        ]
        for c in copies:
            c.wait()
        pltpu.sync_copy(rows_v, out_hbm.at[pl.ds(wid * b_per_w, b_per_w)])

    return gather


def kernel(x, embed):
    b, n, d = x.shape
    flat = x.reshape(-1, d)
    ind, loss = _argmin_and_loss(flat, embed.T)
    quant = _sc_gather(flat.shape[0])(embed, ind.reshape(-1, _IBLK))
    return quant.reshape(b, n, d), ind.reshape(b, n), loss[0, 0]


# speedup vs baseline: 2.4690x; 1.9398x over previous
import functools

import jax
import jax.numpy as jnp
from jax import lax
from jax.experimental import pallas as pl
from jax.experimental.pallas import tpu as pltpu
from jax.experimental.pallas import tpu_sc as plsc

_CB = 8192
_D = 32
_MBLK = 1024
_KBLK = 1024
_NKB = _CB // _KBLK


def _dist_body(xt_ref, e_ref, ind_ref, loss_ref):
    xt = xt_ref[...]
    x2 = jnp.sum(xt * xt, axis=0)

    xbt2 = (2.0 * xt).astype(jnp.bfloat16)

    def chunk(c):
        e_c = e_ref[c * _KBLK:(c + 1) * _KBLK, :]
        s = jnp.dot(e_c.astype(jnp.bfloat16), xbt2,
                    preferred_element_type=jnp.float32)
        e2_c = jnp.sum(e_c * e_c, axis=1)
        q = (x2[None, :] - s) + e2_c[:, None]
        cmin = jnp.min(q, axis=0)
        cidx = jnp.argmin(q, axis=0).astype(jnp.int32) + c * _KBLK
        return cmin, cidx

    def half_argmin(lo, hi):
        run_min, run_idx = chunk(lo)
        for c in range(lo + 1, hi):
            cmin, cidx = chunk(c)
            upd = cmin < run_min
            run_idx = jnp.where(upd, cidx, run_idx)
            run_min = jnp.where(upd, cmin, run_min)
        return run_min, run_idx

    m1, i1 = half_argmin(0, _NKB // 2)
    m2, i2 = half_argmin(_NKB // 2, _NKB)
    m1b = m1.astype(jnp.bfloat16).astype(jnp.float32)
    keep1 = m1b <= m2
    run_idx = jnp.where(keep1, i1, i2)
    run_min = jnp.where(keep1, m1, m2)
    ind_ref[...] = run_idx

    sqd = run_min
    i = pl.program_id(0)
    nprog = pl.num_programs(0)
    prev = jnp.where(i == 0, 0.0, loss_ref[0, 0])
    tot = prev + jnp.sum(sqd)
    denom = jnp.float32(nprog * _MBLK * _D)
    loss_ref[0, 0] = jnp.where(i == nprog - 1, tot / denom, tot)


def _argmin_and_loss(flat_t, embed):
    m = flat_t.shape[1]
    return pl.pallas_call(
        _dist_body,
        grid=(m // _MBLK,),
        in_specs=[
            pl.BlockSpec((_D, _MBLK), lambda i: (0, i)),
            pl.BlockSpec((_CB, _D), lambda i: (0, 0)),
        ],
        out_specs=[
            pl.BlockSpec((_MBLK,), lambda i: (i,)),
            pl.BlockSpec(memory_space=pltpu.SMEM),
        ],
        out_shape=[
            jax.ShapeDtypeStruct((m,), jnp.int32),
            jax.ShapeDtypeStruct((1, 1), jnp.float32),
        ],
    )(flat_t, embed)


_IBLK = 128


@functools.cache
def _sc_gather(b_total):
    info = plsc.get_sparse_core_info()
    nc, ns = info.num_cores, info.num_subcores
    nw = nc * ns
    b_per_w = b_total // nw
    nj = b_per_w // _IBLK
    mesh = plsc.VectorSubcoreMesh(core_axis_name="c", subcore_axis_name="s")

    @functools.partial(
        pl.kernel, mesh=mesh,
        compiler_params=pltpu.CompilerParams(use_tc_tiling_on_sc=False),
        out_type=jax.ShapeDtypeStruct((b_total, _D), jnp.float32),
        scratch_types=[
            pltpu.VMEM((nj, _IBLK), jnp.int32),
            pltpu.VMEM((b_per_w, _D), jnp.float32),
            pltpu.SemaphoreType.DMA,
        ],
    )
    def gather(table_hbm, idx_hbm, out_hbm, idx_v, rows_v, sem):
        wid = lax.axis_index("s") * nc + lax.axis_index("c")
        pltpu.sync_copy(idx_hbm.at[pl.ds(wid * nj, nj)], idx_v)
        copies = [
            pltpu.async_copy(table_hbm.at[idx_v.at[j]],
                             rows_v.at[pl.ds(j * _IBLK, _IBLK)], sem)
            for j in range(nj)
        ]
        for c in copies:
            c.wait()
        pltpu.sync_copy(rows_v, out_hbm.at[pl.ds(wid * b_per_w, b_per_w)])

    return gather


def kernel(x, embed):
    b, n, d = x.shape
    flat = x.reshape(-1, d)
    ind, loss = _argmin_and_loss(flat.T, embed)
    quant = _sc_gather(flat.shape[0])(embed, ind.reshape(-1, _IBLK))
    return quant.reshape(b, n, d), ind.reshape(b, n), loss[0, 0]
